# baseline (speedup 1.0000x reference)
import functools

import jax
import jax.numpy as jnp
from jax import lax
from jax.experimental import pallas as pl
from jax.experimental.pallas import tpu as pltpu
from jax.experimental.pallas import tpu_sc as plsc

NUM_ENT = 1000000
NUM_REL = 1000
EMB_DIM = 32
BATCH = 16384

NC = 2
NS = 16
NW = NC * NS
BPW = BATCH // NW
CHUNK = 128
NCHUNK = BPW // CHUNK
GROUPS = BPW // 16

_mesh = plsc.VectorSubcoreMesh(core_axis_name="c", subcore_axis_name="s")


@functools.partial(
    pl.kernel,
    mesh=_mesh,
    compiler_params=pltpu.CompilerParams(
        needs_layout_passes=False, use_tc_tiling_on_sc=False),
    out_type=jax.ShapeDtypeStruct((BATCH,), jnp.float32),
    scratch_types=[
        pltpu.VMEM((NCHUNK, CHUNK), jnp.int32),
        pltpu.VMEM((NCHUNK, CHUNK), jnp.int32),
        pltpu.VMEM((NCHUNK, CHUNK), jnp.int32),
        pltpu.VMEM((BPW, EMB_DIM), jnp.float32),
        pltpu.VMEM((BPW, EMB_DIM), jnp.float32),
        pltpu.VMEM((BPW, EMB_DIM), jnp.float32),
        pltpu.VMEM((BPW, EMB_DIM), jnp.float32),
        pltpu.VMEM((BPW, EMB_DIM), jnp.float32),
        pltpu.VMEM((BPW, EMB_DIM), jnp.float32),
        pltpu.VMEM((BPW,), jnp.float32),
        pltpu.SemaphoreType.DMA,
    ],
)
def _simple_score(heads_h, rels_h, tails_h, ent_h, ent_t, rel, rel_inv,
                  out_h, idx_h, idx_r, idx_t, hh, ht, th, tt, rv, riv,
                  outv, sem):
    wid = lax.axis_index("s") * NC + lax.axis_index("c")

    pltpu.sync_copy(heads_h.at[wid], idx_h)
    pltpu.sync_copy(rels_h.at[wid], idx_r)
    pltpu.sync_copy(tails_h.at[wid], idx_t)

    copies = []
    for c in range(NCHUNK):
        rows = pl.ds(c * CHUNK, CHUNK)
        copies.append(pltpu.async_copy(ent_h.at[idx_h.at[c]], hh.at[rows], sem))
        copies.append(pltpu.async_copy(ent_h.at[idx_t.at[c]], ht.at[rows], sem))
        copies.append(pltpu.async_copy(ent_t.at[idx_h.at[c]], th.at[rows], sem))
        copies.append(pltpu.async_copy(ent_t.at[idx_t.at[c]], tt.at[rows], sem))
        copies.append(pltpu.async_copy(rel.at[idx_r.at[c]], rv.at[rows], sem))
        copies.append(pltpu.async_copy(rel_inv.at[idx_r.at[c]], riv.at[rows], sem))
    for cp in copies:
        cp.wait()

    lanes = lax.iota(jnp.int32, 16)
    lo = pl.ds(0, 16)
    hi = pl.ds(16, 16)

    def group(g, carry):
        svec = jnp.zeros((16,), jnp.float32)
        for j in range(16):
            b = g * 16 + j
            fwd = (hh[b, lo] * rv[b, lo] * tt[b, lo]
                   + hh[b, hi] * rv[b, hi] * tt[b, hi])
            inv = (ht[b, lo] * riv[b, lo] * th[b, lo]
                   + ht[b, hi] * riv[b, hi] * th[b, hi])
            s = (jnp.sum(fwd) + jnp.sum(inv)) * 0.5
            s = jnp.minimum(jnp.maximum(s, -20.0), 20.0)
            svec = jnp.where(lanes == j, s, svec)
        outv[pl.ds(g * 16, 16)] = svec
        return carry

    lax.fori_loop(0, GROUPS, group, 0)

    pltpu.sync_copy(outv, out_h.at[pl.ds(wid * BPW, BPW)])


def kernel(heads, rels, tails, ent_h_embs, ent_t_embs, rel_embs, rel_inv_embs):
    heads3 = heads.reshape(NW, NCHUNK, CHUNK)
    rels3 = rels.reshape(NW, NCHUNK, CHUNK)
    tails3 = tails.reshape(NW, NCHUNK, CHUNK)
    return _simple_score(heads3, rels3, tails3, ent_h_embs, ent_t_embs,
                         rel_embs, rel_inv_embs)
